# baseline (speedup 1.0000x reference)
import jax
import jax.numpy as jnp
from jax import lax
from jax.experimental import pallas as pl
from jax.experimental.pallas import tpu as pltpu
from jax.experimental.pallas import tpu_sc as plsc

NUM_CORES = 2
NUM_SUBCORES = 16
LANES = 16
NUM_WORKERS = NUM_CORES * NUM_SUBCORES

BATCH = 16384
DIM = 32
NENT = 1000000
BPW = BATCH // NUM_WORKERS
CHUNK = 128
NCHUNK = BPW // CHUNK
GROUPS = BPW // LANES

BLK = 512
NBLK = 63
EPW = NBLK * BLK
TAIL = NENT - 31 * EPW
OPAD = 33


def _transpose_body(ent_t_hbm, out_hbm,
                    in0, in1, out0, out1, tin, tout,
                    sem_i0, sem_i1, sem_o0, sem_o1):
    wid = lax.axis_index("s") * NUM_CORES + lax.axis_index("c")
    base = wid * EPW
    nf = jnp.where(wid < 31, NBLK, 0)
    iota = lax.iota(jnp.int32, LANES)

    ins = [in0, in1]
    outs = [out0, out1]
    isems = [sem_i0, sem_i1]
    osems = [sem_o0, sem_o1]

    def fire_in(b, p):
        eb = pl.multiple_of(base + b * BLK, BLK)
        pltpu.async_copy(ent_t_hbm.at[:, pl.ds(eb, BLK)], ins[p], isems[p])

    def drain_in(p):
        pltpu.make_async_copy(ent_t_hbm.at[:, pl.ds(0, BLK)], ins[p], isems[p]).wait()

    def fire_out(b, p):
        eb = pl.multiple_of(base + b * BLK, BLK)
        pltpu.async_copy(outs[p].at[:, pl.ds(0, DIM)],
                         out_hbm.at[pl.ds(eb, BLK)], osems[p])

    def drain_out(p):
        pltpu.make_async_copy(out_hbm.at[pl.ds(0, BLK)],
                              outs[p].at[:, pl.ds(0, DIM)], osems[p]).wait()

    def transpose_blk(p):
        in_v = ins[p]
        out_v = outs[p]

        def grp(g, carry):
            ent16 = g * LANES + iota
            sl = pl.ds(pl.multiple_of(g * LANES, LANES), LANES)
            for f in range(DIM):
                v = in_v[f, sl]
                fv = jnp.full((LANES,), f, jnp.int32)
                plsc.store_scatter(out_v, [ent16, fv], v)
            return carry

        lax.fori_loop(0, BLK // LANES, grp, 0)

    @pl.when(nf > 0)
    def _():
        fire_in(0, 0)
        fire_in(1, 1)

    def pair(k, carry):
        b0 = k * 2
        b1 = b0 + 1

        @pl.when(b0 < nf)
        def _():
            drain_in(0)

            @pl.when(b0 >= 2)
            def _():
                drain_out(0)

            transpose_blk(0)
            fire_out(b0, 0)

            @pl.when(b0 + 2 < nf)
            def _():
                fire_in(b0 + 2, 0)

        @pl.when(b1 < nf)
        def _():
            drain_in(1)

            @pl.when(b1 >= 2)
            def _():
                drain_out(1)

            transpose_blk(1)
            fire_out(b1, 1)

            @pl.when(b1 + 2 < nf)
            def _():
                fire_in(b1 + 2, 1)

        return carry

    lax.fori_loop(0, (NBLK + 1) // 2, pair, 0)

    @pl.when(nf > 0)
    def _():
        drain_out(0)
        drain_out(1)

    @pl.when(wid == 31)
    def _():
        def strip(s, carry):
            eb = pl.multiple_of(31 * EPW + s * LANES, LANES)
            pltpu.sync_copy(ent_t_hbm.at[:, pl.ds(eb, LANES)], tin)
            for f in range(DIM):
                v = tin[f, pl.ds(0, LANES)]
                fv = jnp.full((LANES,), f, jnp.int32)
                plsc.store_scatter(tout, [iota, fv], v)
            pltpu.sync_copy(tout.at[:, pl.ds(0, DIM)], out_hbm.at[pl.ds(eb, LANES)])
            return carry

        lax.fori_loop(0, TAIL // LANES, strip, 0)


def _score_body(h_hbm, r_hbm, t_hbm, ent_hbm, rel_hbm, out_hbm,
                h_v, r_v, t_v, he_v, re_v, te_v, out_v,
                sem_h, sem_r, sem_t):
    wid = lax.axis_index("s") * NUM_CORES + lax.axis_index("c")
    base = wid * BPW
    pltpu.sync_copy(h_hbm.at[pl.ds(base, BPW)], h_v)
    pltpu.sync_copy(r_hbm.at[pl.ds(base, BPW)], r_v)
    pltpu.sync_copy(t_hbm.at[pl.ds(base, BPW)], t_v)

    copies = []
    for c in range(NCHUNK):
        src = pl.ds(c * CHUNK, CHUNK)
        dst = pl.ds(c * CHUNK, CHUNK)
        copies.append(pltpu.async_copy(ent_hbm.at[h_v.at[src]], he_v.at[dst], sem_h))
        copies.append(pltpu.async_copy(rel_hbm.at[r_v.at[src]], re_v.at[dst], sem_r))
        copies.append(pltpu.async_copy(ent_hbm.at[t_v.at[src]], te_v.at[dst], sem_t))
    for cp in copies:
        cp.wait()

    iota = lax.iota(jnp.int32, LANES)

    def group(g, carry):
        row = iota + g * LANES
        acc = jnp.zeros((LANES,), jnp.float32)
        for j in range(DIM):
            col = lax.rem(iota + j, DIM)
            he = plsc.load_gather(he_v, [row, col])
            re = plsc.load_gather(re_v, [row, col])
            te = plsc.load_gather(te_v, [row, col])
            d = he + re - te
            acc = acc + d * d
        x = acc + 1e-12
        i = plsc.bitcast(x, jnp.int32)
        i = jnp.int32(0x5F3759DF) - (i >> 1)
        y = plsc.bitcast(i, jnp.float32)
        for _ in range(3):
            y = y * (1.5 - 0.5 * x * y * y)
        out_v[pl.ds(pl.multiple_of(g * LANES, LANES), LANES)] = -(x * y)
        return carry

    lax.fori_loop(0, GROUPS, group, 0)
    pltpu.sync_copy(out_v, out_hbm.at[pl.ds(base, BPW)])


def kernel(h, r, t, ent_emb, rel_emb):
    h = h.astype(jnp.int32)
    r = r.astype(jnp.int32)
    t = t.astype(jnp.int32)
    mesh = plsc.VectorSubcoreMesh(core_axis_name="c", subcore_axis_name="s")
    params = pltpu.CompilerParams(
        needs_layout_passes=False, use_tc_tiling_on_sc=False
    )

    transpose_fn = pl.kernel(
        _transpose_body,
        mesh=mesh,
        compiler_params=params,
        out_type=jax.ShapeDtypeStruct((NENT, DIM), jnp.float32),
        scratch_types=[
            pltpu.VMEM((DIM, BLK), jnp.float32),
            pltpu.VMEM((DIM, BLK), jnp.float32),
            pltpu.VMEM((BLK, OPAD), jnp.float32),
            pltpu.VMEM((BLK, OPAD), jnp.float32),
            pltpu.VMEM((DIM, LANES), jnp.float32),
            pltpu.VMEM((LANES, OPAD), jnp.float32),
            pltpu.SemaphoreType.DMA,
            pltpu.SemaphoreType.DMA,
            pltpu.SemaphoreType.DMA,
            pltpu.SemaphoreType.DMA,
        ],
    )
    ent_rm = transpose_fn(ent_emb.T)

    score_fn = pl.kernel(
        _score_body,
        mesh=mesh,
        compiler_params=params,
        out_type=jax.ShapeDtypeStruct((BATCH,), jnp.float32),
        scratch_types=[
            pltpu.VMEM((BPW,), jnp.int32),
            pltpu.VMEM((BPW,), jnp.int32),
            pltpu.VMEM((BPW,), jnp.int32),
            pltpu.VMEM((BPW, DIM), jnp.float32),
            pltpu.VMEM((BPW, DIM), jnp.float32),
            pltpu.VMEM((BPW, DIM), jnp.float32),
            pltpu.VMEM((BPW,), jnp.float32),
            pltpu.SemaphoreType.DMA,
            pltpu.SemaphoreType.DMA,
            pltpu.SemaphoreType.DMA,
        ],
    )
    return score_fn(h, r, t, ent_rm, rel_emb)
